# baseline (speedup 1.0000x reference)
import functools

import jax
import jax.numpy as jnp
from jax import lax
from jax.experimental import pallas as pl
from jax.experimental.pallas import tpu as pltpu
from jax.experimental.pallas import tpu_sc as plsc

B = 16384
NW = 32
BPW = B // NW
NG = BPW // 16
NCH = 16
U_RES = 400
A_RES = 50

_SC_PARAMS = pltpu.CompilerParams(
    needs_layout_passes=False, use_tc_tiling_on_sc=False)


def _sc_repack(u_t):
    mesh = plsc.VectorSubcoreMesh(core_axis_name="c", subcore_axis_name="s")

    @functools.partial(
        pl.kernel,
        out_type=jax.ShapeDtypeStruct((1600, 8, 128), jnp.float32),
        scratch_types=[
            pltpu.VMEM((50, 8, 128), jnp.float32),
            pltpu.VMEM((50, 8, 128), jnp.float32),
            pltpu.SemaphoreType.DMA,
            pltpu.SemaphoreType.DMA,
        ],
        mesh=mesh,
        compiler_params=pltpu.CompilerParams(
            needs_layout_passes=False, use_tc_tiling_on_sc=True),
    )
    def k(u_hbm, uo, ubuf, obuf, sin, sout):
        wid = lax.axis_index("s") * 2 + lax.axis_index("c")
        lanes = lax.iota(jnp.int32, 16)
        rowv = [(16 * b + lanes) >> 4 for b in range(8)]
        colb = [((16 * b + lanes) & 15) * 8 for b in range(8)]

        def src_of(i):
            u = wid * 50 + i
            return u_hbm.at[u >> 2, :, pl.ds((u & 3) * 128, 128)]

        def fire_in(i, _):
            pltpu.async_copy(src_of(i), ubuf.at[i], sin)
            return 0

        def chunk(ch, _):
            i0 = ch * 10
            for i in range(10):
                pltpu.make_async_copy(
                    src_of(i0 + i), ubuf.at[i0 + i], sin).wait()
            for c in range(8):
                colc = [colb[b] + c for b in range(8)]
                for i in range(10):
                    for b in range(8):
                        vals = ubuf[i0 + i, c, pl.ds(16 * b, 16)]
                        plsc.store_scatter(
                            obuf.at[i0 + i], [rowv[b], colc[b]], vals)
            for i in range(10):
                pltpu.async_copy(obuf.at[i0 + i],
                                 uo.at[wid * 50 + i0 + i], sout)
            return 0

        def drain_out(i, _):
            pltpu.make_async_copy(obuf.at[i], uo.at[wid * 50 + i],
                                  sout).wait()
            return 0

        with jax.named_scope("rp_u"):
            lax.fori_loop(0, 50, fire_in, 0)
            lax.fori_loop(0, 5, chunk, 0)
            lax.fori_loop(0, 50, drain_out, 0)

    return k(u_t)


def _sc_gather_features(xT, u_tab, h_t, d_t):
    mesh = plsc.VectorSubcoreMesh(core_axis_name="c", subcore_axis_name="s")

    @functools.partial(
        pl.kernel,
        out_type=jax.ShapeDtypeStruct((NW, 4, 24, 128), jnp.float32),
        scratch_types=[
            pltpu.VMEM((6, BPW), jnp.float32),
            pltpu.VMEM((NCH, 128), jnp.int32),
            pltpu.VMEM((NCH, 128), jnp.int32),
            pltpu.VMEM((NCH, 128), jnp.int32),
            pltpu.VMEM((12, BPW), jnp.float32),
            pltpu.VMEM((4 * BPW, 8), jnp.float32),
            pltpu.VMEM((A_RES, 8, A_RES), jnp.float32),
            pltpu.VMEM((A_RES, 8, A_RES), jnp.float32),
            pltpu.VMEM((4, 24, 128), jnp.float32),
            pltpu.SemaphoreType.DMA,
            pltpu.SemaphoreType.DMA,
        ],
        mesh=mesh,
        compiler_params=_SC_PARAMS,
    )
    def k(x_hbm, u_hbm, h_hbm, d_hbm, out_hbm,
          x_loc, uidx, hidx, didx, wbuf, u_rows, h_loc, d_loc, feat_loc,
          sem, sem2):
        wid = lax.axis_index("s") * 2 + lax.axis_index("c")
        base = wid * BPW
        xcps = [pltpu.async_copy(x_hbm.at[c, pl.ds(base, BPW)],
                                 x_loc.at[c], sem2) for c in range(6)]
        hcp = pltpu.async_copy(h_hbm, h_loc, sem2)
        dcp = pltpu.async_copy(d_hbm, d_loc, sem2)
        for cp in xcps:
            cp.wait()
        lanes = lax.iota(jnp.int32, 16)

        def clamp01(t):
            return jnp.minimum(jnp.maximum(t, 0.0), 1.0)

        def write_idx(buf, corner, qb, rows):
            e = corner * BPW + qb
            buf[e >> 7, pl.ds(e & 127, 16)] = rows

        def idx_u(g, _):
            qb = g * 16
            u = clamp01(x_loc[0, pl.ds(qb, 16)]) * float(U_RES - 1)
            v = clamp01(x_loc[1, pl.ds(qb, 16)]) * float(U_RES - 1)
            x0 = u.astype(jnp.int32)
            y0 = v.astype(jnp.int32)
            fx = u - x0.astype(jnp.float32)
            fy = v - y0.astype(jnp.float32)
            x1 = jnp.minimum(x0 + 1, U_RES - 1)
            y1 = jnp.minimum(y0 + 1, U_RES - 1)
            yr0 = y0 * 512
            yr1 = y1 * 512
            write_idx(uidx, 0, qb, yr0 + x0)
            write_idx(uidx, 1, qb, yr0 + x1)
            write_idx(uidx, 2, qb, yr1 + x0)
            write_idx(uidx, 3, qb, yr1 + x1)
            wbuf[0, pl.ds(qb, 16)] = (1 - fx) * (1 - fy)
            wbuf[1, pl.ds(qb, 16)] = fx * (1 - fy)
            wbuf[2, pl.ds(qb, 16)] = (1 - fx) * fy
            wbuf[3, pl.ds(qb, 16)] = fx * fy
            return 0

        def idx_hd(g, _):
            qb = g * 16

            def wrap_plane(buf, wrow, uc, vc):
                fu = uc - uc.astype(jnp.int32).astype(jnp.float32)
                uw = fu * float(A_RES)
                x0i = uw.astype(jnp.int32)
                ur = uw - x0i.astype(jnp.float32)
                xw0 = jnp.where(x0i >= A_RES, x0i - A_RES, x0i)
                xw1 = jnp.where(xw0 + 1 >= A_RES, xw0 + 1 - A_RES, xw0 + 1)
                vw = clamp01(vc) * float(A_RES - 1)
                yw0 = vw.astype(jnp.int32)
                vr = vw - yw0.astype(jnp.float32)
                yw1 = jnp.minimum(yw0 + 1, A_RES - 1)
                write_idx(buf, 0, qb, yw0 * 512 + xw0)
                write_idx(buf, 1, qb, yw0 * 512 + xw1)
                write_idx(buf, 2, qb, yw1 * 512 + xw0)
                write_idx(buf, 3, qb, yw1 * 512 + xw1)
                wbuf[wrow + 0, pl.ds(qb, 16)] = (1 - ur) * (1 - vr)
                wbuf[wrow + 1, pl.ds(qb, 16)] = ur * (1 - vr)
                wbuf[wrow + 2, pl.ds(qb, 16)] = (1 - ur) * vr
                wbuf[wrow + 3, pl.ds(qb, 16)] = ur * vr

            wrap_plane(hidx, 4, x_loc[3, pl.ds(qb, 16)],
                       x_loc[2, pl.ds(qb, 16)])
            wrap_plane(didx, 8, x_loc[5, pl.ds(qb, 16)],
                       x_loc[4, pl.ds(qb, 16)])
            return 0

        def comp_u(g, _):
            qb = g * 16
            e0 = qb + lanes
            jblk = g >> 3
            off = qb & 127
            acc = [None] * 8
            for corner in range(4):
                w = wbuf[corner, pl.ds(qb, 16)]
                ev = corner * BPW + e0
                for c in range(8):
                    val = plsc.load_gather(
                        u_rows, [ev, jnp.full((16,), c, jnp.int32)])
                    t = w * val
                    acc[c] = t if corner == 0 else acc[c] + t
            for c in range(8):
                feat_loc[jblk, c, pl.ds(off, 16)] = acc[c]
            return 0

        def comp_hd(g, _):
            qb = g * 16
            jblk = g >> 3
            off = qb & 127
            for p, (buf, loc) in enumerate(((hidx, h_loc), (didx, d_loc))):
                acc = [None] * 8
                for corner in range(4):
                    w = wbuf[4 * (p + 1) + corner, pl.ds(qb, 16)]
                    e = corner * BPW + qb
                    ev = buf[e >> 7, pl.ds(e & 127, 16)]
                    yv = ev >> 9
                    xv = ev & 511
                    for c in range(8):
                        val = plsc.load_gather(
                            loc, [yv, jnp.full((16,), c, jnp.int32), xv])
                        t = w * val
                        acc[c] = t if corner == 0 else acc[c] + t
                for c in range(8):
                    feat_loc[jblk, (p + 1) * 8 + c, pl.ds(off, 16)] = acc[c]
            return 0

        with jax.named_scope("g_idxu"):
            lax.fori_loop(0, NG, idx_u, 0)
        with jax.named_scope("g_fire"):
            ucps = [pltpu.async_copy(
                u_hbm.at[uidx.at[j]], u_rows.at[pl.ds(j * 128, 128)], sem)
                for j in range(NCH)]
        with jax.named_scope("g_idxhd"):
            lax.fori_loop(0, NG, idx_hd, 0)
        with jax.named_scope("g_wait_tab"):
            hcp.wait()
            dcp.wait()
        with jax.named_scope("g_redhd"):
            lax.fori_loop(0, NG, comp_hd, 0)
        with jax.named_scope("g_drain"):
            for cp in ucps:
                cp.wait()
        with jax.named_scope("g_redu"):
            lax.fori_loop(0, NG, comp_u, 0)
        with jax.named_scope("g_out"):
            pltpu.sync_copy(feat_loc, out_hbm.at[wid])

    return k(xT, u_tab, h_t, d_t)


def _tc_mlp(feat, W0, W1, W2, W3):
    WB = 8

    def body(f_ref, w0_ref, w1_ref, w2_ref, w3_ref, o_ref):
        w0, w1, w2, w3 = w0_ref[...], w1_ref[...], w2_ref[...], w3_ref[...]
        f = jnp.concatenate(
            [f_ref[wloc, j] for wloc in range(WB) for j in range(4)],
            axis=1)
        h = jnp.maximum(jax.lax.dot(
            w0, f, preferred_element_type=jnp.float32), 0.0)
        h = jnp.maximum(jax.lax.dot(
            w1, h, preferred_element_type=jnp.float32), 0.0)
        h = jnp.maximum(jax.lax.dot(
            w2, h, preferred_element_type=jnp.float32), 0.0)
        o_ref[...] = jax.lax.dot(
            w3, h, preferred_element_type=jnp.float32)

    return pl.pallas_call(
        body,
        grid=(NW // WB,),
        in_specs=[
            pl.BlockSpec((WB, 4, 24, 128), lambda i: (i, 0, 0, 0)),
            pl.BlockSpec((32, 24), lambda i: (0, 0)),
            pl.BlockSpec((32, 32), lambda i: (0, 0)),
            pl.BlockSpec((32, 32), lambda i: (0, 0)),
            pl.BlockSpec((3, 32), lambda i: (0, 0)),
        ],
        out_specs=pl.BlockSpec((3, WB * 4 * 128), lambda i: (0, i)),
        out_shape=jax.ShapeDtypeStruct((3, B), jnp.float32),
    )(feat, W0, W1, W2, W3)


def kernel(x, u_plane, h_plane, d_plane, W0, W1, W2, W3):
    u_t = jnp.transpose(u_plane, (0, 2, 1))
    h_t = jnp.transpose(h_plane, (0, 2, 1))
    d_t = jnp.transpose(d_plane, (0, 2, 1))
    u_tab = _sc_repack(u_t).reshape(400 * 512, 8)
    feat = _sc_gather_features(x.T, u_tab, h_t, d_t)
    out3 = _tc_mlp(feat, W0, W1, W2, W3)
    return out3.T


# speedup vs baseline: 1.0791x; 1.0791x over previous
import functools

import jax
import jax.numpy as jnp
from jax import lax
from jax.experimental import pallas as pl
from jax.experimental.pallas import tpu as pltpu
from jax.experimental.pallas import tpu_sc as plsc

B = 16384
NW = 32
BPW = B // NW
NG = BPW // 16
NCH = 16
U_RES = 400
A_RES = 50

_SC_PARAMS = pltpu.CompilerParams(
    needs_layout_passes=False, use_tc_tiling_on_sc=False)


def _sc_repack(u_t):
    mesh = plsc.VectorSubcoreMesh(core_axis_name="c", subcore_axis_name="s")

    @functools.partial(
        pl.kernel,
        out_type=jax.ShapeDtypeStruct((1600, 8, 128), jnp.float32),
        scratch_types=[
            pltpu.VMEM((50, 8, 128), jnp.float32),
            pltpu.VMEM((50, 8, 128), jnp.float32),
            pltpu.SemaphoreType.DMA,
            pltpu.SemaphoreType.DMA,
        ],
        mesh=mesh,
        compiler_params=pltpu.CompilerParams(
            needs_layout_passes=False, use_tc_tiling_on_sc=True),
    )
    def k(u_hbm, uo, ubuf, obuf, sin, sout):
        wid = lax.axis_index("s") * 2 + lax.axis_index("c")
        lanes = lax.iota(jnp.int32, 16)
        rowv = [(16 * b + lanes) >> 4 for b in range(8)]
        colb = [((16 * b + lanes) & 15) * 8 for b in range(8)]

        def src_of(i):
            u = wid * 50 + i
            return u_hbm.at[u >> 2, :, pl.ds((u & 3) * 128, 128)]

        def fire_in(i, _):
            pltpu.async_copy(src_of(i), ubuf.at[i], sin)
            return 0

        def unit(i, _):
            u = wid * 50 + i
            pltpu.make_async_copy(src_of(i), ubuf.at[i], sin).wait()
            for c in range(8):
                for b in range(8):
                    vals = ubuf[i, c, pl.ds(16 * b, 16)]
                    plsc.store_scatter(
                        obuf.at[i], [rowv[b], colb[b] + c], vals)
            pltpu.async_copy(obuf.at[i], uo.at[u], sout)
            return 0

        def drain_out(i, _):
            pltpu.make_async_copy(obuf.at[i], uo.at[wid * 50 + i],
                                  sout).wait()
            return 0

        with jax.named_scope("rp_u"):
            lax.fori_loop(0, 50, fire_in, 0)
            lax.fori_loop(0, 50, unit, 0)
            lax.fori_loop(0, 50, drain_out, 0)

    return k(u_t)


def _sc_gather_features(xT, u_tab, h_t, d_t):
    mesh = plsc.VectorSubcoreMesh(core_axis_name="c", subcore_axis_name="s")

    @functools.partial(
        pl.kernel,
        out_type=jax.ShapeDtypeStruct((NW, 4, 24, 128), jnp.float32),
        scratch_types=[
            pltpu.VMEM((6, BPW), jnp.float32),
            pltpu.VMEM((NCH, 128), jnp.int32),
            pltpu.VMEM((NCH, 128), jnp.int32),
            pltpu.VMEM((NCH, 128), jnp.int32),
            pltpu.VMEM((12, BPW), jnp.float32),
            pltpu.VMEM((4 * BPW, 8), jnp.float32),
            pltpu.VMEM((A_RES, 8, A_RES), jnp.float32),
            pltpu.VMEM((A_RES, 8, A_RES), jnp.float32),
            pltpu.VMEM((4, 24, 128), jnp.float32),
            pltpu.SemaphoreType.DMA,
            pltpu.SemaphoreType.DMA,
        ],
        mesh=mesh,
        compiler_params=_SC_PARAMS,
    )
    def k(x_hbm, u_hbm, h_hbm, d_hbm, out_hbm,
          x_loc, uidx, hidx, didx, wbuf, u_rows, h_loc, d_loc, feat_loc,
          sem, sem2):
        wid = lax.axis_index("s") * 2 + lax.axis_index("c")
        base = wid * BPW
        xcps = [pltpu.async_copy(x_hbm.at[c, pl.ds(base, BPW)],
                                 x_loc.at[c], sem2) for c in range(6)]
        hcp = pltpu.async_copy(h_hbm, h_loc, sem2)
        dcp = pltpu.async_copy(d_hbm, d_loc, sem2)
        for cp in xcps:
            cp.wait()
        lanes = lax.iota(jnp.int32, 16)

        def clamp01(t):
            return jnp.minimum(jnp.maximum(t, 0.0), 1.0)

        def write_idx(buf, corner, qb, rows):
            e = corner * BPW + qb
            buf[e >> 7, pl.ds(e & 127, 16)] = rows

        def idx_u(g, _):
            qb = g * 16
            u = clamp01(x_loc[0, pl.ds(qb, 16)]) * float(U_RES - 1)
            v = clamp01(x_loc[1, pl.ds(qb, 16)]) * float(U_RES - 1)
            x0 = u.astype(jnp.int32)
            y0 = v.astype(jnp.int32)
            fx = u - x0.astype(jnp.float32)
            fy = v - y0.astype(jnp.float32)
            x1 = jnp.minimum(x0 + 1, U_RES - 1)
            y1 = jnp.minimum(y0 + 1, U_RES - 1)
            yr0 = y0 * 512
            yr1 = y1 * 512
            write_idx(uidx, 0, qb, yr0 + x0)
            write_idx(uidx, 1, qb, yr0 + x1)
            write_idx(uidx, 2, qb, yr1 + x0)
            write_idx(uidx, 3, qb, yr1 + x1)
            wbuf[0, pl.ds(qb, 16)] = (1 - fx) * (1 - fy)
            wbuf[1, pl.ds(qb, 16)] = fx * (1 - fy)
            wbuf[2, pl.ds(qb, 16)] = (1 - fx) * fy
            wbuf[3, pl.ds(qb, 16)] = fx * fy
            return 0

        def idx_hd(g, _):
            qb = g * 16

            def wrap_plane(buf, wrow, uc, vc):
                fu = uc - uc.astype(jnp.int32).astype(jnp.float32)
                uw = fu * float(A_RES)
                x0i = uw.astype(jnp.int32)
                ur = uw - x0i.astype(jnp.float32)
                xw0 = jnp.where(x0i >= A_RES, x0i - A_RES, x0i)
                xw1 = jnp.where(xw0 + 1 >= A_RES, xw0 + 1 - A_RES, xw0 + 1)
                vw = clamp01(vc) * float(A_RES - 1)
                yw0 = vw.astype(jnp.int32)
                vr = vw - yw0.astype(jnp.float32)
                yw1 = jnp.minimum(yw0 + 1, A_RES - 1)
                write_idx(buf, 0, qb, yw0 * 512 + xw0)
                write_idx(buf, 1, qb, yw0 * 512 + xw1)
                write_idx(buf, 2, qb, yw1 * 512 + xw0)
                write_idx(buf, 3, qb, yw1 * 512 + xw1)
                wbuf[wrow + 0, pl.ds(qb, 16)] = (1 - ur) * (1 - vr)
                wbuf[wrow + 1, pl.ds(qb, 16)] = ur * (1 - vr)
                wbuf[wrow + 2, pl.ds(qb, 16)] = (1 - ur) * vr
                wbuf[wrow + 3, pl.ds(qb, 16)] = ur * vr

            wrap_plane(hidx, 4, x_loc[3, pl.ds(qb, 16)],
                       x_loc[2, pl.ds(qb, 16)])
            wrap_plane(didx, 8, x_loc[5, pl.ds(qb, 16)],
                       x_loc[4, pl.ds(qb, 16)])
            return 0

        def comp_u(g, _):
            qb = g * 16
            e0 = qb + lanes
            jblk = g >> 3
            off = qb & 127
            acc = [None] * 8
            for corner in range(4):
                w = wbuf[corner, pl.ds(qb, 16)]
                ev = corner * BPW + e0
                for c in range(8):
                    val = plsc.load_gather(
                        u_rows, [ev, jnp.full((16,), c, jnp.int32)])
                    t = w * val
                    acc[c] = t if corner == 0 else acc[c] + t
            for c in range(8):
                feat_loc[jblk, c, pl.ds(off, 16)] = acc[c]
            return 0

        def comp_hd(g, _):
            qb = g * 16
            jblk = g >> 3
            off = qb & 127
            for p, (buf, loc) in enumerate(((hidx, h_loc), (didx, d_loc))):
                acc = [None] * 8
                for corner in range(4):
                    w = wbuf[4 * (p + 1) + corner, pl.ds(qb, 16)]
                    e = corner * BPW + qb
                    ev = buf[e >> 7, pl.ds(e & 127, 16)]
                    yv = ev >> 9
                    xv = ev & 511
                    for c in range(8):
                        val = plsc.load_gather(
                            loc, [yv, jnp.full((16,), c, jnp.int32), xv])
                        t = w * val
                        acc[c] = t if corner == 0 else acc[c] + t
                for c in range(8):
                    feat_loc[jblk, (p + 1) * 8 + c, pl.ds(off, 16)] = acc[c]
            return 0

        with jax.named_scope("g_idxu"):
            lax.fori_loop(0, NG, idx_u, 0)
        with jax.named_scope("g_fire"):
            ucps = [pltpu.async_copy(
                u_hbm.at[uidx.at[j]], u_rows.at[pl.ds(j * 128, 128)], sem)
                for j in range(NCH)]
        with jax.named_scope("g_idxhd"):
            lax.fori_loop(0, NG, idx_hd, 0)
        with jax.named_scope("g_wait_tab"):
            hcp.wait()
            dcp.wait()
        with jax.named_scope("g_redhd"):
            lax.fori_loop(0, NG, comp_hd, 0)
        with jax.named_scope("g_drain"):
            for cp in ucps:
                cp.wait()
        with jax.named_scope("g_redu"):
            lax.fori_loop(0, NG, comp_u, 0)
        with jax.named_scope("g_out"):
            pltpu.sync_copy(feat_loc, out_hbm.at[wid])

    return k(xT, u_tab, h_t, d_t)


def _tc_mlp(feat, W0, W1, W2, W3):
    WB = 8

    def body(f_ref, w0_ref, w1_ref, w2_ref, w3_ref, o_ref):
        w0, w1, w2, w3 = w0_ref[...], w1_ref[...], w2_ref[...], w3_ref[...]
        f = jnp.concatenate(
            [f_ref[wloc, j] for wloc in range(WB) for j in range(4)],
            axis=1)
        h = jnp.maximum(jax.lax.dot(
            w0, f, preferred_element_type=jnp.float32), 0.0)
        h = jnp.maximum(jax.lax.dot(
            w1, h, preferred_element_type=jnp.float32), 0.0)
        h = jnp.maximum(jax.lax.dot(
            w2, h, preferred_element_type=jnp.float32), 0.0)
        o_ref[...] = jax.lax.dot(
            w3, h, preferred_element_type=jnp.float32)

    return pl.pallas_call(
        body,
        grid=(NW // WB,),
        in_specs=[
            pl.BlockSpec((WB, 4, 24, 128), lambda i: (i, 0, 0, 0)),
            pl.BlockSpec((32, 24), lambda i: (0, 0)),
            pl.BlockSpec((32, 32), lambda i: (0, 0)),
            pl.BlockSpec((32, 32), lambda i: (0, 0)),
            pl.BlockSpec((3, 32), lambda i: (0, 0)),
        ],
        out_specs=pl.BlockSpec((3, WB * 4 * 128), lambda i: (0, i)),
        out_shape=jax.ShapeDtypeStruct((3, B), jnp.float32),
    )(feat, W0, W1, W2, W3)


def kernel(x, u_plane, h_plane, d_plane, W0, W1, W2, W3):
    u_t = jnp.transpose(u_plane, (0, 2, 1))
    h_t = jnp.transpose(h_plane, (0, 2, 1))
    d_t = jnp.transpose(d_plane, (0, 2, 1))
    u_tab = _sc_repack(u_t).reshape(400 * 512, 8)
    feat = _sc_gather_features(x.T, u_tab, h_t, d_t)
    out3 = _tc_mlp(feat, W0, W1, W2, W3)
    return out3.T
